# Initial kernel scaffold; baseline (speedup 1.0000x reference)
import jax
import jax.numpy as jnp
from jax.experimental import pallas as pl


def kernel(query, prompt_pool, prompt_key):
    raise NotImplementedError("write your pallas kernel here")


# speedup vs baseline: 1.5447x; 1.5447x over previous
import functools

import jax
import jax.numpy as jnp
from jax import lax
from jax.experimental import pallas as pl
from jax.experimental.pallas import tpu as pltpu
from jax.experimental.pallas import tpu_sc as plsc

POOL = 8192
PLEN = 8
EMB = 768
BATCH = 1024
K = 5

_BB = 128
_CH = 512
_NCH = POOL // _CH

_NW = 32
_ROWS = BATCH * K
_BPW = _ROWS // _NW
_GCH = 8
_NGCH = _BPW // _GCH


def _knorm_body(k_ref, o_ref):
    k = k_ref[...]
    s = jnp.sum(k * k, axis=1, keepdims=True)
    o_ref[...] = (k / jnp.maximum(jnp.sqrt(s), 1e-12)).astype(jnp.bfloat16)


def _knorm(prompt_key):
    return pl.pallas_call(
        _knorm_body,
        grid=(POOL // 512,),
        in_specs=[pl.BlockSpec((512, EMB), lambda i: (i, 0))],
        out_specs=pl.BlockSpec((512, EMB), lambda i: (i, 0)),
        out_shape=jax.ShapeDtypeStruct((POOL, EMB), jnp.bfloat16),
    )(prompt_key)


def _simtopk_body(q_ref, kn_ref, sim_ref, tkv_ref, tki_ref):
    q = q_ref[...]
    qs = jnp.sum(q * q, axis=1, keepdims=True)
    qn = (q / jnp.maximum(jnp.sqrt(qs), 1e-12)).astype(jnp.bfloat16)

    d_list, i_list, s_list = [], [], []
    for c in range(_NCH):
        kc = kn_ref[pl.ds(c * _CH, _CH), :]
        s = lax.dot_general(qn, kc, (((1,), (1,)), ((), ())),
                            preferred_element_type=jnp.float32)
        sim_ref[:, pl.ds(c * _CH, _CH)] = s
        d = 1.0 - s
        cols = lax.broadcasted_iota(jnp.int32, (_BB, _CH), 1) + c * _CH
        for _ in range(K):
            m = jnp.min(d, axis=1, keepdims=True)
            gi = jnp.min(jnp.where(d == m, cols, jnp.int32(2**30)),
                         axis=1, keepdims=True)
            sel = cols == gi
            sv = jnp.max(jnp.where(sel, s, -jnp.inf), axis=1, keepdims=True)
            d = jnp.where(sel, jnp.inf, d)
            d_list.append(m)
            i_list.append(gi)
            s_list.append(sv)

    cd = jnp.concatenate(d_list, axis=1)
    ci = jnp.concatenate(i_list, axis=1)
    cs = jnp.concatenate(s_list, axis=1)
    outv, outi = [], []
    for _ in range(K):
        m = jnp.min(cd, axis=1, keepdims=True)
        gi = jnp.min(jnp.where(cd == m, ci, jnp.int32(2**30)),
                     axis=1, keepdims=True)
        sel = ci == gi
        sv = jnp.max(jnp.where(sel, cs, -jnp.inf), axis=1, keepdims=True)
        cd = jnp.where(sel, jnp.inf, cd)
        outv.append(sv)
        outi.append(gi)
    tkv_ref[...] = jnp.concatenate(outv + [outv[-1]] * 3, axis=1)
    tki_ref[...] = jnp.concatenate(outi + [outi[-1]] * 3, axis=1)


def _simtopk(query, kn):
    return pl.pallas_call(
        _simtopk_body,
        grid=(BATCH // _BB,),
        in_specs=[
            pl.BlockSpec((_BB, EMB), lambda i: (i, 0)),
            pl.BlockSpec((POOL, EMB), lambda i: (0, 0)),
        ],
        out_specs=[
            pl.BlockSpec((_BB, POOL), lambda i: (i, 0)),
            pl.BlockSpec((_BB, 8), lambda i: (i, 0)),
            pl.BlockSpec((_BB, 8), lambda i: (i, 0)),
        ],
        out_shape=[
            jax.ShapeDtypeStruct((BATCH, POOL), jnp.float32),
            jax.ShapeDtypeStruct((BATCH, 8), jnp.float32),
            jax.ShapeDtypeStruct((BATCH, 8), jnp.int32),
        ],
    )(query, kn)


def _sc_gather_body(idx_hbm, table_hbm, out_hbm, idx_v, buf0, buf1, sem0, sem1):
    wid = lax.axis_index("s") * 2 + lax.axis_index("c")
    base = wid * _BPW
    pltpu.sync_copy(idx_hbm.at[pl.ds(base, _BPW)], idx_v)
    bufs = (buf0, buf1)
    sems = (sem0, sem1)

    def start(g):
        return pltpu.async_copy(
            table_hbm.at[idx_v.at[pl.ds(g * _GCH, _GCH)]],
            bufs[g % 2], sems[g % 2])

    h = start(0)
    for g in range(_NGCH):
        h.wait()
        if g + 1 < _NGCH:
            h = start(g + 1)
        pltpu.sync_copy(bufs[g % 2], out_hbm.at[pl.ds(base + g * _GCH, _GCH)])


@functools.partial(jax.jit)
def _sc_gather(idx_flat, table):
    mesh = plsc.VectorSubcoreMesh(core_axis_name="c", subcore_axis_name="s")
    f = functools.partial(
        pl.kernel,
        mesh=mesh,
        out_type=jax.ShapeDtypeStruct((_ROWS, PLEN * EMB), jnp.float32),
        scratch_types=[
            pltpu.VMEM((_BPW,), jnp.int32),
            pltpu.VMEM((_GCH, PLEN * EMB), jnp.float32),
            pltpu.VMEM((_GCH, PLEN * EMB), jnp.float32),
            pltpu.SemaphoreType.DMA,
            pltpu.SemaphoreType.DMA,
        ],
    )(_sc_gather_body)
    return f(idx_flat, table)


def kernel(query, prompt_pool, prompt_key):
    kn = _knorm(prompt_key)
    sim, tkv8, tki8 = _simtopk(query, kn)
    tkv = tkv8[:, :K]
    idx_flat = tki8[:, :K].reshape(-1)
    table = prompt_pool.reshape(POOL, PLEN * EMB)
    sel = _sc_gather(idx_flat, table)
    return sel.reshape(BATCH, K * PLEN, EMB), sim, tkv


# speedup vs baseline: 2.3594x; 1.5274x over previous
import functools

import jax
import jax.numpy as jnp
from jax import lax
from jax.experimental import pallas as pl
from jax.experimental.pallas import tpu as pltpu
from jax.experimental.pallas import tpu_sc as plsc

POOL = 8192
PLEN = 8
EMB = 768
BATCH = 1024
K = 5

_BB = 128
_CH = 512
_NCH = POOL // _CH

_NW = 32
_ROWS = BATCH * K
_BPW = _ROWS // _NW
_GCH = 8
_NGCH = _BPW // _GCH


def _simtopk_body(q_ref, kn_ref, sim_ref, tkv_ref, tki_ref):
    qn = q_ref[...]

    d_list, i_list, s_list = [], [], []
    for c in range(_NCH):
        kc = kn_ref[pl.ds(c * _CH, _CH), :]
        s = lax.dot_general(qn, kc, (((1,), (1,)), ((), ())),
                            preferred_element_type=jnp.float32)
        sim_ref[:, pl.ds(c * _CH, _CH)] = s
        d = 1.0 - s
        cols = lax.broadcasted_iota(jnp.int32, (_BB, _CH), 1) + c * _CH
        for _ in range(K):
            m = jnp.min(d, axis=1, keepdims=True)
            gi = jnp.min(jnp.where(d == m, cols, jnp.int32(2**30)),
                         axis=1, keepdims=True)
            sel = cols == gi
            sv = jnp.max(jnp.where(sel, s, -jnp.inf), axis=1, keepdims=True)
            d = jnp.where(sel, jnp.inf, d)
            d_list.append(m)
            i_list.append(gi)
            s_list.append(sv)

    cd = jnp.concatenate(d_list, axis=1)
    ci = jnp.concatenate(i_list, axis=1)
    cs = jnp.concatenate(s_list, axis=1)
    outv, outi = [], []
    for _ in range(K):
        m = jnp.min(cd, axis=1, keepdims=True)
        gi = jnp.min(jnp.where(cd == m, ci, jnp.int32(2**30)),
                     axis=1, keepdims=True)
        sel = ci == gi
        sv = jnp.max(jnp.where(sel, cs, -jnp.inf), axis=1, keepdims=True)
        cd = jnp.where(sel, jnp.inf, cd)
        outv.append(sv)
        outi.append(gi)
    tkv_ref[...] = jnp.concatenate(outv + [outv[-1]] * 3, axis=1)
    tki_ref[...] = jnp.concatenate(outi + [outi[-1]] * 3, axis=1)


def _simtopk(qn, kn):
    return pl.pallas_call(
        _simtopk_body,
        grid=(BATCH // _BB,),
        in_specs=[
            pl.BlockSpec((_BB, EMB), lambda i: (i, 0)),
            pl.BlockSpec((POOL, EMB), lambda i: (0, 0)),
        ],
        out_specs=[
            pl.BlockSpec((_BB, POOL), lambda i: (i, 0)),
            pl.BlockSpec((_BB, 8), lambda i: (i, 0)),
            pl.BlockSpec((_BB, 8), lambda i: (i, 0)),
        ],
        out_shape=[
            jax.ShapeDtypeStruct((BATCH, POOL), jnp.float32),
            jax.ShapeDtypeStruct((BATCH, 8), jnp.float32),
            jax.ShapeDtypeStruct((BATCH, 8), jnp.int32),
        ],
    )(qn, kn)


def _sc_gather_body(idx_hbm, table_hbm, out_hbm, idx_v, buf0, buf1, sem0, sem1):
    wid = lax.axis_index("s") * 2 + lax.axis_index("c")
    base = wid * _BPW
    pltpu.sync_copy(idx_hbm.at[pl.ds(base, _BPW)], idx_v)
    bufs = (buf0, buf1)
    sems = (sem0, sem1)

    def start(g):
        return pltpu.async_copy(
            table_hbm.at[idx_v.at[pl.ds(g * _GCH, _GCH)]],
            bufs[g % 2], sems[g % 2])

    h = start(0)
    for g in range(_NGCH):
        h.wait()
        if g + 1 < _NGCH:
            h = start(g + 1)
        pltpu.sync_copy(bufs[g % 2], out_hbm.at[pl.ds(base + g * _GCH, _GCH)])


@functools.partial(jax.jit)
def _sc_gather(idx_flat, table):
    mesh = plsc.VectorSubcoreMesh(core_axis_name="c", subcore_axis_name="s")
    f = functools.partial(
        pl.kernel,
        mesh=mesh,
        out_type=jax.ShapeDtypeStruct((_ROWS, PLEN, EMB), jnp.float32),
        scratch_types=[
            pltpu.VMEM((_BPW,), jnp.int32),
            pltpu.VMEM((_GCH, PLEN, EMB), jnp.float32),
            pltpu.VMEM((_GCH, PLEN, EMB), jnp.float32),
            pltpu.SemaphoreType.DMA,
            pltpu.SemaphoreType.DMA,
        ],
    )(_sc_gather_body)
    return f(idx_flat, table)


def _l2n(x):
    n = jnp.sqrt(jnp.sum(x * x, axis=1, keepdims=True))
    return (x / jnp.maximum(n, 1e-12)).astype(jnp.bfloat16)


def kernel(query, prompt_pool, prompt_key):
    sim, tkv8, tki8 = _simtopk(_l2n(query), _l2n(prompt_key))
    tkv = tkv8[:, :K]
    idx_flat = tki8[:, :K].reshape(-1)
    sel = _sc_gather(idx_flat, prompt_pool)
    return sel.reshape(BATCH, K * PLEN, EMB), sim, tkv


# speedup vs baseline: 2.7670x; 1.1728x over previous
import functools

import jax
import jax.numpy as jnp
from jax import lax
from jax.experimental import pallas as pl
from jax.experimental.pallas import tpu as pltpu
from jax.experimental.pallas import tpu_sc as plsc

POOL = 8192
PLEN = 8
EMB = 768
BATCH = 1024
K = 5

_BB = 128
_CH = 512
_NCH = POOL // _CH

_NW = 32
_ROWS = BATCH * K
_BPW = _ROWS // _NW
_GCH = 8
_NGCH = _BPW // _GCH


def _topk_lex(d, cols, k):
    big = jnp.int32(2**30)
    ms, gs = [], []
    m_prev = g_prev = None
    for t in range(k):
        if t == 0:
            md = d
        else:
            active = (d > m_prev) | ((d == m_prev) & (cols > g_prev))
            md = jnp.where(active, d, jnp.inf)
        m = jnp.min(md, axis=1, keepdims=True)
        gi = jnp.min(jnp.where(md == m, cols, big), axis=1, keepdims=True)
        m_prev, g_prev = m, gi
        ms.append(m)
        gs.append(gi)
    return ms, gs


def _simtopk_body(q_ref, kn_ref, sim_ref, tkv_ref, tki_ref):
    qn = q_ref[...]

    cols = lax.broadcasted_iota(jnp.int32, (_BB, _CH), 1)
    d_list, i_list = [], []
    for c in range(_NCH):
        kc = kn_ref[pl.ds(c * _CH, _CH), :]
        s = lax.dot_general(qn, kc, (((1,), (1,)), ((), ())),
                            preferred_element_type=jnp.float32)
        sim_ref[:, pl.ds(c * _CH, _CH)] = s
        ms, gs = _topk_lex(1.0 - s, cols, K)
        d_list += ms
        i_list += [g + c * _CH for g in gs]

    cd = jnp.concatenate(d_list, axis=1)
    ci = jnp.concatenate(i_list, axis=1)
    ms, gs = _topk_lex(cd, ci, K)
    outv = [1.0 - m for m in ms]
    tkv_ref[...] = jnp.concatenate(outv + [outv[-1]] * 3, axis=1)
    tki_ref[...] = jnp.concatenate(gs + [gs[-1]] * 3, axis=1)


def _simtopk(qn, kn):
    return pl.pallas_call(
        _simtopk_body,
        grid=(BATCH // _BB,),
        in_specs=[
            pl.BlockSpec((_BB, EMB), lambda i: (i, 0)),
            pl.BlockSpec((POOL, EMB), lambda i: (0, 0)),
        ],
        out_specs=[
            pl.BlockSpec((_BB, POOL), lambda i: (i, 0)),
            pl.BlockSpec((_BB, 8), lambda i: (i, 0)),
            pl.BlockSpec((_BB, 8), lambda i: (i, 0)),
        ],
        out_shape=[
            jax.ShapeDtypeStruct((BATCH, POOL), jnp.float32),
            jax.ShapeDtypeStruct((BATCH, 8), jnp.float32),
            jax.ShapeDtypeStruct((BATCH, 8), jnp.int32),
        ],
    )(qn, kn)


def _sc_gather_body(idx_hbm, table_hbm, out_hbm, idx_v, buf0, buf1, sem0, sem1):
    wid = lax.axis_index("s") * 2 + lax.axis_index("c")
    base = wid * _BPW
    pltpu.sync_copy(idx_hbm.at[pl.ds(base, _BPW)], idx_v)
    bufs = (buf0, buf1)
    sems = (sem0, sem1)

    def start(g):
        return pltpu.async_copy(
            table_hbm.at[idx_v.at[pl.ds(g * _GCH, _GCH)]],
            bufs[g % 2], sems[g % 2])

    h = start(0)
    for g in range(_NGCH):
        h.wait()
        if g + 1 < _NGCH:
            h = start(g + 1)
        pltpu.sync_copy(bufs[g % 2], out_hbm.at[pl.ds(base + g * _GCH, _GCH)])


@functools.partial(jax.jit)
def _sc_gather(idx_flat, table):
    mesh = plsc.VectorSubcoreMesh(core_axis_name="c", subcore_axis_name="s")
    f = functools.partial(
        pl.kernel,
        mesh=mesh,
        out_type=jax.ShapeDtypeStruct((_ROWS, PLEN, EMB), jnp.float32),
        scratch_types=[
            pltpu.VMEM((_BPW,), jnp.int32),
            pltpu.VMEM((_GCH, PLEN, EMB), jnp.float32),
            pltpu.VMEM((_GCH, PLEN, EMB), jnp.float32),
            pltpu.SemaphoreType.DMA,
            pltpu.SemaphoreType.DMA,
        ],
    )(_sc_gather_body)
    return f(idx_flat, table)


def _l2n(x):
    n = jnp.sqrt(jnp.sum(x * x, axis=1, keepdims=True))
    return (x / jnp.maximum(n, 1e-12)).astype(jnp.bfloat16)


def kernel(query, prompt_pool, prompt_key):
    sim, tkv8, tki8 = _simtopk(_l2n(query), _l2n(prompt_key))
    tkv = tkv8[:, :K]
    idx_flat = tki8[:, :K].reshape(-1)
    sel = _sc_gather(idx_flat, prompt_pool)
    return sel.reshape(BATCH, K * PLEN, EMB), sim, tkv


# speedup vs baseline: 2.9629x; 1.0708x over previous
import functools

import jax
import jax.numpy as jnp
from jax import lax
from jax.experimental import pallas as pl
from jax.experimental.pallas import tpu as pltpu
from jax.experimental.pallas import tpu_sc as plsc

POOL = 8192
PLEN = 8
EMB = 768
BATCH = 1024
K = 5

_BB = 128
_CH = 512
_NCH = POOL // _CH
_HB = BATCH // 2

_NW = 32
_ROWS = BATCH * K
_HROWS = _HB * K
_BPW = _HROWS // _NW
_GCH = 8
_NGCH = _BPW // _GCH


def _topk_lex(d, cols, k):
    big = jnp.int32(2**30)
    ms, gs = [], []
    m_prev = g_prev = None
    for t in range(k):
        if t == 0:
            md = d
        else:
            active = (d > m_prev) | ((d == m_prev) & (cols > g_prev))
            md = jnp.where(active, d, jnp.inf)
        m = jnp.min(md, axis=1, keepdims=True)
        gi = jnp.min(jnp.where(md == m, cols, big), axis=1, keepdims=True)
        m_prev, g_prev = m, gi
        ms.append(m)
        gs.append(gi)
    return ms, gs


def _simtopk_body(q_ref, kn_ref, sim_ref, tkv_ref, tki_ref):
    qn = q_ref[...]

    cols = lax.broadcasted_iota(jnp.int32, (_BB, _CH), 1)
    d_list, i_list = [], []
    for c in range(_NCH):
        kc = kn_ref[pl.ds(c * _CH, _CH), :]
        s = lax.dot_general(qn, kc, (((1,), (1,)), ((), ())),
                            preferred_element_type=jnp.float32)
        sim_ref[:, pl.ds(c * _CH, _CH)] = s
        ms, gs = _topk_lex(1.0 - s, cols, K)
        d_list += ms
        i_list += [g + c * _CH for g in gs]

    cd = jnp.concatenate(d_list, axis=1)
    ci = jnp.concatenate(i_list, axis=1)
    ms, gs = _topk_lex(cd, ci, K)
    outv = [1.0 - m for m in ms]
    tkv_ref[...] = jnp.concatenate(outv + [outv[-1]] * 3, axis=1)
    tki_ref[...] = jnp.concatenate(gs + [gs[-1]] * 3, axis=1)


def _simtopk(qn, kn):
    nb = qn.shape[0]
    return pl.pallas_call(
        _simtopk_body,
        grid=(nb // _BB,),
        in_specs=[
            pl.BlockSpec((_BB, EMB), lambda i: (i, 0)),
            pl.BlockSpec((POOL, EMB), lambda i: (0, 0)),
        ],
        out_specs=[
            pl.BlockSpec((_BB, POOL), lambda i: (i, 0)),
            pl.BlockSpec((_BB, 8), lambda i: (i, 0)),
            pl.BlockSpec((_BB, 8), lambda i: (i, 0)),
        ],
        out_shape=[
            jax.ShapeDtypeStruct((nb, POOL), jnp.float32),
            jax.ShapeDtypeStruct((nb, 8), jnp.float32),
            jax.ShapeDtypeStruct((nb, 8), jnp.int32),
        ],
    )(qn, kn)


def _sc_gather_rows(idx_hbm, table_hbm, out_hbm, idx_v, bufs, sems, row0):
    wid = lax.axis_index("s") * 2 + lax.axis_index("c")
    base = wid * _BPW
    pltpu.sync_copy(idx_hbm.at[pl.ds(base, _BPW)], idx_v)

    def start(g):
        return pltpu.async_copy(
            table_hbm.at[idx_v.at[pl.ds(g * _GCH, _GCH)]],
            bufs[g % 2], sems[g % 2])

    h = start(0)
    for g in range(_NGCH):
        h.wait()
        if g + 1 < _NGCH:
            h = start(g + 1)
        pltpu.sync_copy(bufs[g % 2],
                        out_hbm.at[pl.ds(row0 + base + g * _GCH, _GCH)])


def _sc_body0(idx_hbm, table_hbm, out_hbm, idx_v, buf0, buf1, sem0, sem1):
    _sc_gather_rows(idx_hbm, table_hbm, out_hbm, idx_v,
                    (buf0, buf1), (sem0, sem1), 0)


def _sc_body1(idx_hbm, table_hbm, out_ref, idx_v, buf0, buf1, sem0, sem1):
    _sc_gather_rows(idx_hbm, table_hbm, out_ref, idx_v,
                    (buf0, buf1), (sem0, sem1), _HROWS)


_SC_SCRATCH = [
    pltpu.VMEM((_BPW,), jnp.int32),
    pltpu.VMEM((_GCH, PLEN, EMB), jnp.float32),
    pltpu.VMEM((_GCH, PLEN, EMB), jnp.float32),
    pltpu.SemaphoreType.DMA,
    pltpu.SemaphoreType.DMA,
]


def _sc_mesh():
    return plsc.VectorSubcoreMesh(core_axis_name="c", subcore_axis_name="s")


def _sc_gather0(idx_flat, table):
    f = functools.partial(
        pl.kernel,
        mesh=_sc_mesh(),
        out_type=jax.ShapeDtypeStruct((_ROWS, PLEN, EMB), jnp.float32),
        scratch_types=_SC_SCRATCH,
    )(_sc_body0)
    return f(idx_flat, table)


def _sc_gather1(idx_flat, table, out_ref):
    f = functools.partial(
        pl.kernel,
        mesh=_sc_mesh(),
        out_type=(),
        scratch_types=_SC_SCRATCH,
    )(_sc_body1)
    return f(idx_flat, table, out_ref)


def _l2n(x):
    n = jnp.sqrt(jnp.sum(x * x, axis=1, keepdims=True))
    return (x / jnp.maximum(n, 1e-12)).astype(jnp.bfloat16)


def kernel(query, prompt_pool, prompt_key):
    qn = _l2n(query)
    kn = _l2n(prompt_key)
    sim1, tkv1, tki1 = _simtopk(qn[:_HB], kn)
    sim2, tkv2, tki2 = _simtopk(qn[_HB:], kn)
    sel0 = _sc_gather0(tki1[:, :K].reshape(-1), prompt_pool)
    out_ref = jax.new_ref(sel0)
    _sc_gather1(tki2[:, :K].reshape(-1), prompt_pool, out_ref)
    sel = out_ref[...]
    sim = jnp.concatenate([sim1, sim2], axis=0)
    tkv = jnp.concatenate([tkv1[:, :K], tkv2[:, :K]], axis=0)
    return sel.reshape(BATCH, K * PLEN, EMB), sim, tkv


# speedup vs baseline: 2.9912x; 1.0096x over previous
import functools

import jax
import jax.numpy as jnp
from jax import lax
from jax.experimental import pallas as pl
from jax.experimental.pallas import tpu as pltpu
from jax.experimental.pallas import tpu_sc as plsc

POOL = 8192
PLEN = 8
EMB = 768
BATCH = 1024
K = 5

_BB = 128
_CH = 1024
_NCH = POOL // _CH
_HB = BATCH // 2

_NW = 32
_ROWS = BATCH * K
_HROWS = _HB * K
_BPW = _HROWS // _NW
_GCH = 8
_NGCH = _BPW // _GCH


def _topk_lex(d, cols, k):
    big = jnp.int32(2**30)
    ms, gs = [], []
    m_prev = g_prev = None
    for t in range(k):
        if t == 0:
            md = d
        else:
            active = (d > m_prev) | ((d == m_prev) & (cols > g_prev))
            md = jnp.where(active, d, jnp.inf)
        m = jnp.min(md, axis=1, keepdims=True)
        gi = jnp.min(jnp.where(md == m, cols, big), axis=1, keepdims=True)
        m_prev, g_prev = m, gi
        ms.append(m)
        gs.append(gi)
    return ms, gs


def _simtopk_body(q_ref, kn_ref, sim_ref, tkv_ref, tki_ref):
    qn = q_ref[...]

    cols = lax.broadcasted_iota(jnp.int32, (_BB, _CH), 1)
    d_list, i_list = [], []
    for c in range(_NCH):
        kc = kn_ref[pl.ds(c * _CH, _CH), :]
        s = lax.dot_general(qn, kc, (((1,), (1,)), ((), ())),
                            preferred_element_type=jnp.float32)
        sim_ref[:, pl.ds(c * _CH, _CH)] = s
        ms, gs = _topk_lex(1.0 - s, cols, K)
        d_list += ms
        i_list += [g + c * _CH for g in gs]

    cd = jnp.concatenate(d_list, axis=1)
    ci = jnp.concatenate(i_list, axis=1)
    ms, gs = _topk_lex(cd, ci, K)
    outv = [1.0 - m for m in ms]
    tkv_ref[...] = jnp.concatenate(outv + [outv[-1]] * 3, axis=1)
    tki_ref[...] = jnp.concatenate(gs + [gs[-1]] * 3, axis=1)


def _simtopk(qn, kn):
    nb = qn.shape[0]
    return pl.pallas_call(
        _simtopk_body,
        grid=(nb // _BB,),
        in_specs=[
            pl.BlockSpec((_BB, EMB), lambda i: (i, 0)),
            pl.BlockSpec((POOL, EMB), lambda i: (0, 0)),
        ],
        out_specs=[
            pl.BlockSpec((_BB, POOL), lambda i: (i, 0)),
            pl.BlockSpec((_BB, 8), lambda i: (i, 0)),
            pl.BlockSpec((_BB, 8), lambda i: (i, 0)),
        ],
        out_shape=[
            jax.ShapeDtypeStruct((nb, POOL), jnp.float32),
            jax.ShapeDtypeStruct((nb, 8), jnp.float32),
            jax.ShapeDtypeStruct((nb, 8), jnp.int32),
        ],
    )(qn, kn)


def _sc_gather_rows(idx_hbm, table_hbm, out_hbm, idx_v, bufs, sems, row0):
    wid = lax.axis_index("s") * 2 + lax.axis_index("c")
    base = wid * _BPW
    pltpu.sync_copy(idx_hbm.at[pl.ds(base, _BPW)], idx_v)

    def start(g):
        return pltpu.async_copy(
            table_hbm.at[idx_v.at[pl.ds(g * _GCH, _GCH)]],
            bufs[g % 2], sems[g % 2])

    h = start(0)
    for g in range(_NGCH):
        h.wait()
        if g + 1 < _NGCH:
            h = start(g + 1)
        pltpu.sync_copy(bufs[g % 2],
                        out_hbm.at[pl.ds(row0 + base + g * _GCH, _GCH)])


def _sc_body0(idx_hbm, table_hbm, out_hbm, idx_v, buf0, buf1, sem0, sem1):
    _sc_gather_rows(idx_hbm, table_hbm, out_hbm, idx_v,
                    (buf0, buf1), (sem0, sem1), 0)


def _sc_body1(idx_hbm, table_hbm, out_ref, idx_v, buf0, buf1, sem0, sem1):
    _sc_gather_rows(idx_hbm, table_hbm, out_ref, idx_v,
                    (buf0, buf1), (sem0, sem1), _HROWS)


_SC_SCRATCH = [
    pltpu.VMEM((_BPW,), jnp.int32),
    pltpu.VMEM((_GCH, PLEN, EMB), jnp.float32),
    pltpu.VMEM((_GCH, PLEN, EMB), jnp.float32),
    pltpu.SemaphoreType.DMA,
    pltpu.SemaphoreType.DMA,
]


def _sc_mesh():
    return plsc.VectorSubcoreMesh(core_axis_name="c", subcore_axis_name="s")


def _sc_gather0(idx_flat, table):
    f = functools.partial(
        pl.kernel,
        mesh=_sc_mesh(),
        out_type=jax.ShapeDtypeStruct((_ROWS, PLEN, EMB), jnp.float32),
        scratch_types=_SC_SCRATCH,
    )(_sc_body0)
    return f(idx_flat, table)


def _sc_gather1(idx_flat, table, out_ref):
    f = functools.partial(
        pl.kernel,
        mesh=_sc_mesh(),
        out_type=(),
        scratch_types=_SC_SCRATCH,
    )(_sc_body1)
    return f(idx_flat, table, out_ref)


def _l2n(x):
    n = jnp.sqrt(jnp.sum(x * x, axis=1, keepdims=True))
    return (x / jnp.maximum(n, 1e-12)).astype(jnp.bfloat16)


def kernel(query, prompt_pool, prompt_key):
    qn = _l2n(query)
    kn = _l2n(prompt_key)
    sim1, tkv1, tki1 = _simtopk(qn[:_HB], kn)
    sim2, tkv2, tki2 = _simtopk(qn[_HB:], kn)
    sel0 = _sc_gather0(tki1[:, :K].reshape(-1), prompt_pool)
    out_ref = jax.new_ref(sel0)
    _sc_gather1(tki2[:, :K].reshape(-1), prompt_pool, out_ref)
    sel = out_ref[...]
    sim = jnp.concatenate([sim1, sim2], axis=0)
    tkv = jnp.concatenate([tkv1[:, :K], tkv2[:, :K]], axis=0)
    return sel.reshape(BATCH, K * PLEN, EMB), sim, tkv


# speedup vs baseline: 3.2633x; 1.0909x over previous
import functools

import jax
import jax.numpy as jnp
from jax import lax
from jax.experimental import pallas as pl
from jax.experimental.pallas import tpu as pltpu
from jax.experimental.pallas import tpu_sc as plsc

POOL = 8192
PLEN = 8
EMB = 768
BATCH = 1024
K = 5

_BB = 128
_CH = 1024
_NCH = POOL // _CH
_HB = BATCH // 2

_NW = 32
_ROWS = BATCH * K
_HROWS = _HB * K
_BPW = _HROWS // _NW
_GCH = 8
_NGCH = _BPW // _GCH


def _topk_lex(d, cols, k):
    big = jnp.int32(2**30)
    ms, gs = [], []
    m_prev = g_prev = None
    for t in range(k):
        if t == 0:
            md = d
        else:
            active = (d > m_prev) | ((d == m_prev) & (cols > g_prev))
            md = jnp.where(active, d, jnp.inf)
        m = jnp.min(md, axis=1, keepdims=True)
        gi = jnp.min(jnp.where(md == m, cols, big), axis=1, keepdims=True)
        m_prev, g_prev = m, gi
        ms.append(m)
        gs.append(gi)
    return ms, gs


def _simtopk_body(q_ref, kn_ref, sim_ref, tkv_ref, tki_ref):
    qn = q_ref[...]

    cols = lax.broadcasted_iota(jnp.int32, (_BB, _CH), 1)
    d_list, i_list = [], []
    for c in range(_NCH):
        kc = kn_ref[pl.ds(c * _CH, _CH), :]
        s = lax.dot_general(qn, kc, (((1,), (1,)), ((), ())),
                            preferred_element_type=jnp.float32)
        sim_ref[:, pl.ds(c * _CH, _CH)] = s
        ms, gs = _topk_lex(1.0 - s, cols, K)
        d_list += ms
        i_list += [g + c * _CH for g in gs]

    cd = jnp.concatenate(d_list, axis=1)
    ci = jnp.concatenate(i_list, axis=1)
    ms, gs = _topk_lex(cd, ci, K)
    outv = [1.0 - m for m in ms]
    tkv_ref[...] = jnp.concatenate(outv + [outv[-1]] * 3, axis=1)
    tki_ref[...] = jnp.concatenate(gs + [gs[-1]] * 3, axis=1)


def _simtopk_first(qn, kn):
    return pl.pallas_call(
        _simtopk_body,
        grid=(_HB // _BB,),
        in_specs=[
            pl.BlockSpec((_BB, EMB), lambda i: (i, 0)),
            pl.BlockSpec((POOL, EMB), lambda i: (0, 0)),
        ],
        out_specs=[
            pl.BlockSpec((_BB, POOL), lambda i: (i, 0)),
            pl.BlockSpec((_BB, 8), lambda i: (i, 0)),
            pl.BlockSpec((_BB, 8), lambda i: (i, 0)),
        ],
        out_shape=[
            jax.ShapeDtypeStruct((BATCH, POOL), jnp.float32),
            jax.ShapeDtypeStruct((_HB, 8), jnp.float32),
            jax.ShapeDtypeStruct((_HB, 8), jnp.int32),
        ],
    )(qn, kn)


def _simtopk_second(qn, kn, sim_in):
    off = _HB // _BB
    return pl.pallas_call(
        lambda q_ref, kn_ref, si_ref, sim_ref, tkv_ref, tki_ref:
            _simtopk_body(q_ref, kn_ref, sim_ref, tkv_ref, tki_ref),
        grid=(_HB // _BB,),
        in_specs=[
            pl.BlockSpec((_BB, EMB), lambda i: (i, 0)),
            pl.BlockSpec((POOL, EMB), lambda i: (0, 0)),
            pl.BlockSpec(memory_space=pl.ANY),
        ],
        out_specs=[
            pl.BlockSpec((_BB, POOL), lambda i: (i + off, 0)),
            pl.BlockSpec((_BB, 8), lambda i: (i, 0)),
            pl.BlockSpec((_BB, 8), lambda i: (i, 0)),
        ],
        out_shape=[
            jax.ShapeDtypeStruct((BATCH, POOL), jnp.float32),
            jax.ShapeDtypeStruct((_HB, 8), jnp.float32),
            jax.ShapeDtypeStruct((_HB, 8), jnp.int32),
        ],
        input_output_aliases={2: 0},
    )(qn, kn, sim_in)


def _sc_gather_rows(idx_hbm, table_hbm, out_hbm, idx_v, bufs, sems, row0):
    wid = lax.axis_index("s") * 2 + lax.axis_index("c")
    base = wid * _BPW
    pltpu.sync_copy(idx_hbm.at[pl.ds(base, _BPW)], idx_v)

    def start(g):
        return pltpu.async_copy(
            table_hbm.at[idx_v.at[pl.ds(g * _GCH, _GCH)]],
            bufs[g % 2], sems[g % 2])

    h = start(0)
    for g in range(_NGCH):
        h.wait()
        if g + 1 < _NGCH:
            h = start(g + 1)
        pltpu.sync_copy(bufs[g % 2],
                        out_hbm.at[pl.ds(row0 + base + g * _GCH, _GCH)])


def _sc_body0(idx_hbm, table_hbm, out_hbm, idx_v, buf0, buf1, sem0, sem1):
    _sc_gather_rows(idx_hbm, table_hbm, out_hbm, idx_v,
                    (buf0, buf1), (sem0, sem1), 0)


def _sc_body1(idx_hbm, table_hbm, out_ref, idx_v, buf0, buf1, sem0, sem1):
    _sc_gather_rows(idx_hbm, table_hbm, out_ref, idx_v,
                    (buf0, buf1), (sem0, sem1), _HROWS)


_SC_SCRATCH = [
    pltpu.VMEM((_BPW,), jnp.int32),
    pltpu.VMEM((_GCH, PLEN, EMB), jnp.float32),
    pltpu.VMEM((_GCH, PLEN, EMB), jnp.float32),
    pltpu.SemaphoreType.DMA,
    pltpu.SemaphoreType.DMA,
]


def _sc_mesh():
    return plsc.VectorSubcoreMesh(core_axis_name="c", subcore_axis_name="s")


def _sc_gather0(idx_flat, table):
    f = functools.partial(
        pl.kernel,
        mesh=_sc_mesh(),
        out_type=jax.ShapeDtypeStruct((_ROWS, PLEN, EMB), jnp.float32),
        scratch_types=_SC_SCRATCH,
    )(_sc_body0)
    return f(idx_flat, table)


def _sc_gather1(idx_flat, table, out_ref):
    f = functools.partial(
        pl.kernel,
        mesh=_sc_mesh(),
        out_type=(),
        scratch_types=_SC_SCRATCH,
    )(_sc_body1)
    return f(idx_flat, table, out_ref)


def _l2n(x):
    n = jnp.sqrt(jnp.sum(x * x, axis=1, keepdims=True))
    return (x / jnp.maximum(n, 1e-12)).astype(jnp.bfloat16)


def kernel(query, prompt_pool, prompt_key):
    qn = _l2n(query)
    kn = _l2n(prompt_key)
    sim1, tkv1, tki1 = _simtopk_first(qn[:_HB], kn)
    sim, tkv2, tki2 = _simtopk_second(qn[_HB:], kn, sim1)
    sel0 = _sc_gather0(tki1[:, :K].reshape(-1), prompt_pool)
    out_ref = jax.new_ref(sel0)
    _sc_gather1(tki2[:, :K].reshape(-1), prompt_pool, out_ref)
    sel = out_ref[...]
    tkv = jnp.concatenate([tkv1[:, :K], tkv2[:, :K]], axis=0)
    return sel.reshape(BATCH, K * PLEN, EMB), sim, tkv


# speedup vs baseline: 3.2637x; 1.0001x over previous
import functools

import jax
import jax.numpy as jnp
from jax import lax
from jax.experimental import pallas as pl
from jax.experimental.pallas import tpu as pltpu
from jax.experimental.pallas import tpu_sc as plsc

POOL = 8192
PLEN = 8
EMB = 768
BATCH = 1024
K = 5

_BB = 128
_CH = 1024
_NCH = POOL // _CH

_NSPLIT = 4
_PB = BATCH // _NSPLIT
_PBLK = _PB // _BB

_NW = 32
_ROWS = BATCH * K
_PROWS = _PB * K
_BPW = _PROWS // _NW
_GCH = 8
_NGCH = _BPW // _GCH


def _topk_lex(d, cols, k):
    big = jnp.int32(2**30)
    ms, gs = [], []
    m_prev = g_prev = None
    for t in range(k):
        if t == 0:
            md = d
        else:
            active = (d > m_prev) | ((d == m_prev) & (cols > g_prev))
            md = jnp.where(active, d, jnp.inf)
        m = jnp.min(md, axis=1, keepdims=True)
        gi = jnp.min(jnp.where(md == m, cols, big), axis=1, keepdims=True)
        m_prev, g_prev = m, gi
        ms.append(m)
        gs.append(gi)
    return ms, gs


def _simtopk_body(q_ref, kn_ref, sim_ref, tkv_ref, tki_ref):
    qn = q_ref[...]

    cols = lax.broadcasted_iota(jnp.int32, (_BB, _CH), 1)
    d_list, i_list = [], []
    for c in range(_NCH):
        kc = kn_ref[pl.ds(c * _CH, _CH), :]
        s = lax.dot_general(qn, kc, (((1,), (1,)), ((), ())),
                            preferred_element_type=jnp.float32)
        sim_ref[:, pl.ds(c * _CH, _CH)] = s
        ms, gs = _topk_lex(1.0 - s, cols, K)
        d_list += ms
        i_list += [g + c * _CH for g in gs]

    cd = jnp.concatenate(d_list, axis=1)
    ci = jnp.concatenate(i_list, axis=1)
    ms, gs = _topk_lex(cd, ci, K)
    outv = [1.0 - m for m in ms]
    tkv_ref[...] = jnp.concatenate(outv + [outv[-1]] * 3, axis=1)
    tki_ref[...] = jnp.concatenate(gs + [gs[-1]] * 3, axis=1)


def _simtopk_part(qn_part, kn, sim_in, part):
    off = part * _PBLK
    out_specs = [
        pl.BlockSpec((_BB, POOL), lambda i: (i + off, 0)),
        pl.BlockSpec((_BB, 8), lambda i: (i, 0)),
        pl.BlockSpec((_BB, 8), lambda i: (i, 0)),
    ]
    out_shape = [
        jax.ShapeDtypeStruct((BATCH, POOL), jnp.float32),
        jax.ShapeDtypeStruct((_PB, 8), jnp.float32),
        jax.ShapeDtypeStruct((_PB, 8), jnp.int32),
    ]
    in_specs = [
        pl.BlockSpec((_BB, EMB), lambda i: (i, 0)),
        pl.BlockSpec((POOL, EMB), lambda i: (0, 0)),
    ]
    if part == 0:
        return pl.pallas_call(
            _simtopk_body,
            grid=(_PBLK,),
            in_specs=in_specs,
            out_specs=out_specs,
            out_shape=out_shape,
        )(qn_part, kn)
    return pl.pallas_call(
        lambda q_ref, kn_ref, si_ref, sim_ref, tkv_ref, tki_ref:
            _simtopk_body(q_ref, kn_ref, sim_ref, tkv_ref, tki_ref),
        grid=(_PBLK,),
        in_specs=in_specs + [pl.BlockSpec(memory_space=pl.ANY)],
        out_specs=out_specs,
        out_shape=out_shape,
        input_output_aliases={2: 0},
    )(qn_part, kn, sim_in)


def _sc_gather_rows(idx_hbm, table_hbm, out_hbm, idx_v, bufs, sems, row0):
    wid = lax.axis_index("s") * 2 + lax.axis_index("c")
    base = wid * _BPW
    pltpu.sync_copy(idx_hbm.at[pl.ds(base, _BPW)], idx_v)

    def start(g):
        return pltpu.async_copy(
            table_hbm.at[idx_v.at[pl.ds(g * _GCH, _GCH)]],
            bufs[g % 2], sems[g % 2])

    h = start(0)
    for g in range(_NGCH):
        h.wait()
        if g + 1 < _NGCH:
            h = start(g + 1)
        pltpu.sync_copy(bufs[g % 2],
                        out_hbm.at[pl.ds(row0 + base + g * _GCH, _GCH)])


def _make_sc_body(row0):
    def body(idx_hbm, table_hbm, out, idx_v, buf0, buf1, sem0, sem1):
        _sc_gather_rows(idx_hbm, table_hbm, out, idx_v,
                        (buf0, buf1), (sem0, sem1), row0)
    return body


_SC_SCRATCH = [
    pltpu.VMEM((_BPW,), jnp.int32),
    pltpu.VMEM((_GCH, PLEN, EMB), jnp.float32),
    pltpu.VMEM((_GCH, PLEN, EMB), jnp.float32),
    pltpu.SemaphoreType.DMA,
    pltpu.SemaphoreType.DMA,
]


def _sc_mesh():
    return plsc.VectorSubcoreMesh(core_axis_name="c", subcore_axis_name="s")


def _sc_gather_first(idx_flat, table):
    f = functools.partial(
        pl.kernel,
        mesh=_sc_mesh(),
        out_type=jax.ShapeDtypeStruct((_ROWS, PLEN, EMB), jnp.float32),
        scratch_types=_SC_SCRATCH,
    )(_make_sc_body(0))
    return f(idx_flat, table)


def _sc_gather_into(idx_flat, table, out_ref, part):
    f = functools.partial(
        pl.kernel,
        mesh=_sc_mesh(),
        out_type=(),
        scratch_types=_SC_SCRATCH,
    )(_make_sc_body(part * _PROWS))
    return f(idx_flat, table, out_ref)


def _l2n(x):
    n = jnp.sqrt(jnp.sum(x * x, axis=1, keepdims=True))
    return (x / jnp.maximum(n, 1e-12)).astype(jnp.bfloat16)


def kernel(query, prompt_pool, prompt_key):
    qn = _l2n(query)
    kn = _l2n(prompt_key)

    sim = None
    tkvs, idxs = [], []
    for p in range(_NSPLIT):
        sim, tkv, tki = _simtopk_part(qn[p * _PB:(p + 1) * _PB], kn, sim, p)
        tkvs.append(tkv[:, :K])
        idxs.append(tki[:, :K].reshape(-1))

    sel0 = _sc_gather_first(idxs[0], prompt_pool)
    out_ref = jax.new_ref(sel0)
    for p in range(1, _NSPLIT):
        _sc_gather_into(idxs[p], prompt_pool, out_ref, p)
    sel = out_ref[...]

    tkv = jnp.concatenate(tkvs, axis=0)
    return sel.reshape(BATCH, K * PLEN, EMB), sim, tkv


# speedup vs baseline: 3.3685x; 1.0321x over previous
import functools

import jax
import jax.numpy as jnp
from jax import lax
from jax.experimental import pallas as pl
from jax.experimental.pallas import tpu as pltpu
from jax.experimental.pallas import tpu_sc as plsc

POOL = 8192
PLEN = 8
EMB = 768
BATCH = 1024
K = 5

_BB = 256
_CH = 1024
_NCH = POOL // _CH

_NSPLIT = 4
_PB = BATCH // _NSPLIT
_PBLK = _PB // _BB

_NW = 32
_ROWS = BATCH * K
_PROWS = _PB * K
_BPW = _PROWS // _NW
_GCH = 8
_NGCH = _BPW // _GCH


def _topk_lex(d, cols, k):
    big = jnp.int32(2**30)
    ms, gs = [], []
    m_prev = g_prev = None
    for t in range(k):
        if t == 0:
            md = d
        else:
            active = (d > m_prev) | ((d == m_prev) & (cols > g_prev))
            md = jnp.where(active, d, jnp.inf)
        m = jnp.min(md, axis=1, keepdims=True)
        gi = jnp.min(jnp.where(md == m, cols, big), axis=1, keepdims=True)
        m_prev, g_prev = m, gi
        ms.append(m)
        gs.append(gi)
    return ms, gs


def _simtopk_body(q_ref, kn_ref, sim_ref, tkv_ref, tki_ref):
    qn = q_ref[...]

    cols = lax.broadcasted_iota(jnp.int32, (_BB, _CH), 1)
    d_list, i_list = [], []
    for c in range(_NCH):
        kc = kn_ref[pl.ds(c * _CH, _CH), :]
        s = lax.dot_general(qn, kc, (((1,), (1,)), ((), ())),
                            preferred_element_type=jnp.float32)
        sim_ref[:, pl.ds(c * _CH, _CH)] = s
        ms, gs = _topk_lex(1.0 - s, cols, K)
        d_list += ms
        i_list += [g + c * _CH for g in gs]

    cd = jnp.concatenate(d_list, axis=1)
    ci = jnp.concatenate(i_list, axis=1)
    ms, gs = _topk_lex(cd, ci, K)
    outv = [1.0 - m for m in ms]
    tkv_ref[...] = jnp.concatenate(outv + [outv[-1]] * 3, axis=1)
    tki_ref[...] = jnp.concatenate(gs + [gs[-1]] * 3, axis=1)


def _simtopk_part(qn_part, kn, sim_in, part):
    off = part * _PBLK
    out_specs = [
        pl.BlockSpec((_BB, POOL), lambda i: (i + off, 0)),
        pl.BlockSpec((_BB, 8), lambda i: (i, 0)),
        pl.BlockSpec((_BB, 8), lambda i: (i, 0)),
    ]
    out_shape = [
        jax.ShapeDtypeStruct((BATCH, POOL), jnp.float32),
        jax.ShapeDtypeStruct((_PB, 8), jnp.float32),
        jax.ShapeDtypeStruct((_PB, 8), jnp.int32),
    ]
    in_specs = [
        pl.BlockSpec((_BB, EMB), lambda i: (i, 0)),
        pl.BlockSpec((POOL, EMB), lambda i: (0, 0)),
    ]
    if part == 0:
        return pl.pallas_call(
            _simtopk_body,
            grid=(_PBLK,),
            in_specs=in_specs,
            out_specs=out_specs,
            out_shape=out_shape,
        )(qn_part, kn)
    return pl.pallas_call(
        lambda q_ref, kn_ref, si_ref, sim_ref, tkv_ref, tki_ref:
            _simtopk_body(q_ref, kn_ref, sim_ref, tkv_ref, tki_ref),
        grid=(_PBLK,),
        in_specs=in_specs + [pl.BlockSpec(memory_space=pl.ANY)],
        out_specs=out_specs,
        out_shape=out_shape,
        input_output_aliases={2: 0},
    )(qn_part, kn, sim_in)


def _sc_gather_rows(idx_hbm, table_hbm, out_hbm, idx_v, bufs, sems, row0):
    wid = lax.axis_index("s") * 2 + lax.axis_index("c")
    base = wid * _BPW
    pltpu.sync_copy(idx_hbm.at[pl.ds(base, _BPW)], idx_v)

    def start(g):
        return pltpu.async_copy(
            table_hbm.at[idx_v.at[pl.ds(g * _GCH, _GCH)]],
            bufs[g % 2], sems[g % 2])

    h = start(0)
    for g in range(_NGCH):
        h.wait()
        if g + 1 < _NGCH:
            h = start(g + 1)
        pltpu.sync_copy(bufs[g % 2],
                        out_hbm.at[pl.ds(row0 + base + g * _GCH, _GCH)])


def _make_sc_body(row0):
    def body(idx_hbm, table_hbm, out, idx_v, buf0, buf1, sem0, sem1):
        _sc_gather_rows(idx_hbm, table_hbm, out, idx_v,
                        (buf0, buf1), (sem0, sem1), row0)
    return body


_SC_SCRATCH = [
    pltpu.VMEM((_BPW,), jnp.int32),
    pltpu.VMEM((_GCH, PLEN, EMB), jnp.float32),
    pltpu.VMEM((_GCH, PLEN, EMB), jnp.float32),
    pltpu.SemaphoreType.DMA,
    pltpu.SemaphoreType.DMA,
]


def _sc_mesh():
    return plsc.VectorSubcoreMesh(core_axis_name="c", subcore_axis_name="s")


def _sc_gather_first(idx_flat, table):
    f = functools.partial(
        pl.kernel,
        mesh=_sc_mesh(),
        out_type=jax.ShapeDtypeStruct((_ROWS, PLEN, EMB), jnp.float32),
        scratch_types=_SC_SCRATCH,
    )(_make_sc_body(0))
    return f(idx_flat, table)


def _sc_gather_into(idx_flat, table, out_ref, part):
    f = functools.partial(
        pl.kernel,
        mesh=_sc_mesh(),
        out_type=(),
        scratch_types=_SC_SCRATCH,
    )(_make_sc_body(part * _PROWS))
    return f(idx_flat, table, out_ref)


def _l2n(x):
    n = jnp.sqrt(jnp.sum(x * x, axis=1, keepdims=True))
    return (x / jnp.maximum(n, 1e-12)).astype(jnp.bfloat16)


def kernel(query, prompt_pool, prompt_key):
    qn = _l2n(query)
    kn = _l2n(prompt_key)

    sim = None
    tkvs, idxs = [], []
    for p in range(_NSPLIT):
        sim, tkv, tki = _simtopk_part(qn[p * _PB:(p + 1) * _PB], kn, sim, p)
        tkvs.append(tkv[:, :K])
        idxs.append(tki[:, :K].reshape(-1))

    sel0 = _sc_gather_first(idxs[0], prompt_pool)
    out_ref = jax.new_ref(sel0)
    for p in range(1, _NSPLIT):
        _sc_gather_into(idxs[p], prompt_pool, out_ref, p)
    sel = out_ref[...]

    tkv = jnp.concatenate(tkvs, axis=0)
    return sel.reshape(BATCH, K * PLEN, EMB), sim, tkv


# speedup vs baseline: 3.4416x; 1.0217x over previous
import functools

import jax
import jax.numpy as jnp
from jax import lax
from jax.experimental import pallas as pl
from jax.experimental.pallas import tpu as pltpu
from jax.experimental.pallas import tpu_sc as plsc

POOL = 8192
PLEN = 8
EMB = 768
BATCH = 1024
K = 5

_BB = 256
_CH = 1024
_NCH = POOL // _CH

_NSPLIT = 4
_PB = BATCH // _NSPLIT
_PBLK = _PB // _BB

_NW = 32
_ROWS = BATCH * K
_PROWS = _PB * K
_BPW = _PROWS // _NW
_GCH = 8
_NGCH = _BPW // _GCH


def _topk_lex(d, cols, k):
    big = jnp.int32(2**30)
    ms, gs = [], []
    m_prev = g_prev = None
    for t in range(k):
        if t == 0:
            md = d
        else:
            active = (d > m_prev) | ((d == m_prev) & (cols > g_prev))
            md = jnp.where(active, d, jnp.inf)
        m = jnp.min(md, axis=1, keepdims=True)
        gi = jnp.min(jnp.where(md == m, cols, big), axis=1, keepdims=True)
        m_prev, g_prev = m, gi
        ms.append(m)
        gs.append(gi)
    return ms, gs


_KSTEPS = 2
_KHALF = POOL // _KSTEPS
_NCHS = _KHALF // _CH
_NCAND = _NCHS * K


def _simtopk_body(q_ref, kn_ref, sim_ref, tkv_ref, tki_ref, cd_ref, ci_ref):
    j = pl.program_id(0)
    qn = q_ref[...]

    cols = lax.broadcasted_iota(jnp.int32, (_BB, _CH), 1)
    d_list, i_list = [], []
    for c in range(_NCHS):
        kc = kn_ref[pl.ds(c * _CH, _CH), :]
        s = lax.dot_general(qn, kc, (((1,), (1,)), ((), ())),
                            preferred_element_type=jnp.float32)
        sim_ref[:, pl.ds(c * _CH, _CH)] = s
        ms, gs = _topk_lex(1.0 - s, cols, K)
        d_list += ms
        i_list += [g + (j * _KHALF + c * _CH) for g in gs]

    cd = jnp.concatenate(d_list, axis=1)
    ci = jnp.concatenate(i_list, axis=1)

    @pl.when(j == 0)
    def _():
        cd_ref[...] = cd
        ci_ref[...] = ci

    @pl.when(j == _KSTEPS - 1)
    def _():
        cda = jnp.concatenate([cd_ref[...], cd], axis=1)
        cia = jnp.concatenate([ci_ref[...], ci], axis=1)
        ms, gs = _topk_lex(cda, cia, K)
        outv = [1.0 - m for m in ms]
        tkv_ref[...] = jnp.concatenate(outv + [outv[-1]] * 3, axis=1)
        tki_ref[...] = jnp.concatenate(gs + [gs[-1]] * 3, axis=1)


def _simtopk_part(qn_part, kn, sim_in, part):
    off = part * _PBLK
    out_specs = [
        pl.BlockSpec((_BB, _KHALF), lambda j: (off, j)),
        pl.BlockSpec((_BB, 8), lambda j: (0, 0)),
        pl.BlockSpec((_BB, 8), lambda j: (0, 0)),
    ]
    out_shape = [
        jax.ShapeDtypeStruct((BATCH, POOL), jnp.float32),
        jax.ShapeDtypeStruct((_PB, 8), jnp.float32),
        jax.ShapeDtypeStruct((_PB, 8), jnp.int32),
    ]
    in_specs = [
        pl.BlockSpec((_BB, EMB), lambda j: (0, 0)),
        pl.BlockSpec((_KHALF, EMB), lambda j: (j, 0)),
    ]
    scratch = [
        pltpu.VMEM((_BB, _NCAND), jnp.float32),
        pltpu.VMEM((_BB, _NCAND), jnp.int32),
    ]
    if part == 0:
        return pl.pallas_call(
            _simtopk_body,
            grid=(_KSTEPS,),
            in_specs=in_specs,
            out_specs=out_specs,
            out_shape=out_shape,
            scratch_shapes=scratch,
        )(qn_part, kn)
    return pl.pallas_call(
        lambda q_ref, kn_ref, si_ref, sim_ref, tkv_ref, tki_ref, cd_ref, ci_ref:
            _simtopk_body(q_ref, kn_ref, sim_ref, tkv_ref, tki_ref,
                          cd_ref, ci_ref),
        grid=(_KSTEPS,),
        in_specs=in_specs + [pl.BlockSpec(memory_space=pl.ANY)],
        out_specs=out_specs,
        out_shape=out_shape,
        scratch_shapes=scratch,
        input_output_aliases={2: 0},
    )(qn_part, kn, sim_in)


def _sc_gather_rows(idx_hbm, table_hbm, out_hbm, idx_v, bufs, sems, row0):
    wid = lax.axis_index("s") * 2 + lax.axis_index("c")
    base = wid * _BPW
    pltpu.sync_copy(idx_hbm.at[pl.ds(base, _BPW)], idx_v)

    def start(g):
        return pltpu.async_copy(
            table_hbm.at[idx_v.at[pl.ds(g * _GCH, _GCH)]],
            bufs[g % 2], sems[g % 2])

    h = start(0)
    for g in range(_NGCH):
        h.wait()
        if g + 1 < _NGCH:
            h = start(g + 1)
        pltpu.sync_copy(bufs[g % 2],
                        out_hbm.at[pl.ds(row0 + base + g * _GCH, _GCH)])


def _make_sc_body(row0):
    def body(idx_hbm, table_hbm, out, idx_v, buf0, buf1, sem0, sem1):
        _sc_gather_rows(idx_hbm, table_hbm, out, idx_v,
                        (buf0, buf1), (sem0, sem1), row0)
    return body


_SC_SCRATCH = [
    pltpu.VMEM((_BPW,), jnp.int32),
    pltpu.VMEM((_GCH, PLEN, EMB), jnp.float32),
    pltpu.VMEM((_GCH, PLEN, EMB), jnp.float32),
    pltpu.SemaphoreType.DMA,
    pltpu.SemaphoreType.DMA,
]


def _sc_mesh():
    return plsc.VectorSubcoreMesh(core_axis_name="c", subcore_axis_name="s")


def _sc_gather_first(idx_flat, table):
    f = functools.partial(
        pl.kernel,
        mesh=_sc_mesh(),
        out_type=jax.ShapeDtypeStruct((_ROWS, PLEN, EMB), jnp.float32),
        scratch_types=_SC_SCRATCH,
    )(_make_sc_body(0))
    return f(idx_flat, table)


def _sc_gather_into(idx_flat, table, out_ref, part):
    f = functools.partial(
        pl.kernel,
        mesh=_sc_mesh(),
        out_type=(),
        scratch_types=_SC_SCRATCH,
    )(_make_sc_body(part * _PROWS))
    return f(idx_flat, table, out_ref)


def _l2n(x):
    n = jnp.sqrt(jnp.sum(x * x, axis=1, keepdims=True))
    return (x / jnp.maximum(n, 1e-12)).astype(jnp.bfloat16)


def kernel(query, prompt_pool, prompt_key):
    qn = _l2n(query)
    kn = _l2n(prompt_key)

    sim = None
    tkvs, idxs = [], []
    for p in range(_NSPLIT):
        sim, tkv, tki = _simtopk_part(qn[p * _PB:(p + 1) * _PB], kn, sim, p)
        tkvs.append(tkv[:, :K])
        idxs.append(tki[:, :K].reshape(-1))

    sel0 = _sc_gather_first(idxs[0], prompt_pool)
    out_ref = jax.new_ref(sel0)
    for p in range(1, _NSPLIT):
        _sc_gather_into(idxs[p], prompt_pool, out_ref, p)
    sel = out_ref[...]

    tkv = jnp.concatenate(tkvs, axis=0)
    return sel.reshape(BATCH, K * PLEN, EMB), sim, tkv
